# baseline (speedup 1.0000x reference)
import jax
import jax.numpy as jnp
from jax.experimental import pallas as pl

B = 8
N = 256
D = 128
BJ = 64
NJ = N // BJ
LAYERS = 2

_SQRT_HALF = 0.7071067811865476


def _gelu(x):
    x3 = x * x * x
    return 0.5 * x * (1.0 + jnp.tanh(0.7978845608028654 * (x + 0.044715 * x3)))


def _dot(x, w):
    return jax.lax.dot_general(x, w, (((1,), (0,)), ((), ())),
                               preferred_element_type=jnp.float32)


def _gnn_kernel(h_ref, coord_ref, lng_ref, lnb_ref, wc1_ref, bc1_ref,
                wc2_ref, bc2_ref, w1a_ref, w1b_ref, w1c_ref, b1_ref,
                w2_ref, b2_ref, wn1a_ref, wn1b_ref, bn1_ref, wn2_ref,
                bn2_ref, out_ref):
    h = h_ref[...]
    c = coord_ref[0]
    cx = c[:, 0:1]
    cy = c[:, 1:2]
    for l in range(LAYERS):
        g = lng_ref[l]
        beta = lnb_ref[l]
        wc1 = wc1_ref[l]
        bc1 = bc1_ref[l]
        wc2 = wc2_ref[l]
        bc2 = bc2_ref[l]
        w1a = w1a_ref[l]
        w1b = w1b_ref[l]
        w1c = w1c_ref[l]
        b1 = b1_ref[l]
        w2 = w2_ref[l]
        b2 = b2_ref[l]
        wn1a = wn1a_ref[l]
        wn1b = wn1b_ref[l]
        bn1 = bn1_ref[l]
        wn2 = wn2_ref[l]
        bn2 = bn2_ref[l]

        mu = jnp.mean(h, axis=1, keepdims=True)
        xc = h - mu
        var = jnp.mean(xc * xc, axis=1, keepdims=True)
        hn = xc * jax.lax.rsqrt(var + 1e-5) * g + beta

        a = _dot(hn, w1a) + b1
        bv = _dot(hn, w1b)
        u_i = cx * wc1[0:1, :] + cy * wc1[1:2, :] + bc1
        u_j = u_i - bc1

        cm0 = _gelu(_dot(_gelu(bc1), wc2) + bc2)
        c0 = _dot(cm0, w1c)
        tii = _gelu(a + bv + c0)
        eii = _gelu(_dot(tii, w2) + b2)

        acc = -eii
        for j in range(NJ):
            uj = u_j[j * BJ:(j + 1) * BJ, :]
            bvj = bv[j * BJ:(j + 1) * BJ, :]
            cm1 = _gelu(u_i[:, None, :] - uj[None, :, :])
            cm = _gelu(_dot(cm1.reshape(N * BJ, D), wc2) + bc2)
            cb = _dot(cm, w1c)
            t3 = cb.reshape(N, BJ, D) + a[:, None, :] + bvj[None, :, :]
            e = _gelu(_dot(_gelu(t3).reshape(N * BJ, D), w2) + b2)
            acc = acc + e.reshape(N, BJ, D).sum(axis=1)

        o1 = _gelu(_dot(hn, wn1a) + _dot(acc, wn1b) + bn1)
        h = _dot(o1, wn2) + bn2
    out_ref[...] = h


def kernel(h, coord, params):
    layers = params["layers"]

    def stack(f):
        return jnp.stack([f(p) for p in layers])

    lng = stack(lambda p: p["ln_g"].reshape(1, D))
    lnb = stack(lambda p: p["ln_b"].reshape(1, D))
    wc1 = stack(lambda p: p["coord1"]["W"])
    bc1 = stack(lambda p: p["coord1"]["b"].reshape(1, D))
    wc2 = stack(lambda p: p["coord2"]["W"])
    bc2 = stack(lambda p: p["coord2"]["b"].reshape(1, D))
    w1a = stack(lambda p: p["edge1"]["W"][0:D])
    w1b = stack(lambda p: p["edge1"]["W"][D:2 * D])
    w1c = stack(lambda p: p["edge1"]["W"][2 * D:3 * D])
    b1 = stack(lambda p: p["edge1"]["b"].reshape(1, D))
    w2 = stack(lambda p: p["edge2"]["W"])
    b2 = stack(lambda p: p["edge2"]["b"].reshape(1, D))
    wn1a = stack(lambda p: p["node1"]["W"][0:D])
    wn1b = stack(lambda p: p["node1"]["W"][D:2 * D])
    bn1 = stack(lambda p: p["node1"]["b"].reshape(1, D))
    wn2 = stack(lambda p: p["node2"]["W"])
    bn2 = stack(lambda p: p["node2"]["b"].reshape(1, D))

    def full(shape):
        return pl.BlockSpec(shape, lambda b: (0,) * len(shape))

    out = pl.pallas_call(
        _gnn_kernel,
        grid=(B,),
        in_specs=[
            pl.BlockSpec((N, D), lambda b: (b, 0)),
            pl.BlockSpec((1, N, 2), lambda b: (b, 0, 0)),
            full((LAYERS, 1, D)), full((LAYERS, 1, D)),
            full((LAYERS, 2, D)), full((LAYERS, 1, D)),
            full((LAYERS, D, D)), full((LAYERS, 1, D)),
            full((LAYERS, D, D)), full((LAYERS, D, D)),
            full((LAYERS, D, D)), full((LAYERS, 1, D)),
            full((LAYERS, D, D)), full((LAYERS, 1, D)),
            full((LAYERS, D, D)), full((LAYERS, D, D)),
            full((LAYERS, 1, D)),
            full((LAYERS, D, D)), full((LAYERS, 1, D)),
        ],
        out_specs=pl.BlockSpec((N, D), lambda b: (b, 0)),
        out_shape=jax.ShapeDtypeStruct((B * N, D), jnp.float32),
    )(h, coord, lng, lnb, wc1, bc1, wc2, bc2, w1a, w1b, w1c, b1, w2, b2,
      wn1a, wn1b, bn1, wn2, bn2)
    return out


# speedup vs baseline: 1.4867x; 1.4867x over previous
import jax
import jax.numpy as jnp
from jax.experimental import pallas as pl

B = 8
N = 256
D = 128
BJ = 128
NJ = N // BJ
LAYERS = 2

_SQRT_HALF = 0.7071067811865476


def _gelu(x):
    return 0.5 * x * (1.0 + jax.lax.erf(x * _SQRT_HALF))


def _dot(x, w):
    return jax.lax.dot_general(x, w, (((1,), (0,)), ((), ())),
                               preferred_element_type=jnp.float32)


def _gnn_kernel(h_ref, coord_ref, lng_ref, lnb_ref, wc1_ref, bc1_ref,
                wc2_ref, bc2_ref, w1a_ref, w1b_ref, w1c_ref, b1_ref,
                w2_ref, b2_ref, wn1a_ref, wn1b_ref, bn1_ref, wn2_ref,
                bn2_ref, out_ref):
    h = h_ref[...]
    c = coord_ref[0]
    cx = c[:, 0:1]
    cy = c[:, 1:2]
    for l in range(LAYERS):
        g = lng_ref[l]
        beta = lnb_ref[l]
        wc1 = wc1_ref[l]
        bc1 = bc1_ref[l]
        wc2 = wc2_ref[l]
        bc2 = bc2_ref[l]
        w1a = w1a_ref[l]
        w1b = w1b_ref[l]
        w1c = w1c_ref[l]
        b1 = b1_ref[l]
        w2 = w2_ref[l]
        b2 = b2_ref[l]
        wn1a = wn1a_ref[l]
        wn1b = wn1b_ref[l]
        bn1 = bn1_ref[l]
        wn2 = wn2_ref[l]
        bn2 = bn2_ref[l]

        mu = jnp.mean(h, axis=1, keepdims=True)
        xc = h - mu
        var = jnp.mean(xc * xc, axis=1, keepdims=True)
        hn = xc * jax.lax.rsqrt(var + 1e-5) * g + beta

        a = _dot(hn, w1a) + b1
        bv = _dot(hn, w1b)
        u_i = cx * wc1[0:1, :] + cy * wc1[1:2, :] + bc1
        u_j = u_i - bc1

        cm0 = _gelu(_dot(_gelu(bc1), wc2) + bc2)
        c0 = _dot(cm0, w1c)
        tii = _gelu(a + bv + c0)
        eii = _gelu(_dot(tii, w2) + b2)

        acc = -eii
        for j in range(NJ):
            uj = u_j[j * BJ:(j + 1) * BJ, :]
            bvj = bv[j * BJ:(j + 1) * BJ, :]
            cm1 = _gelu(u_i[:, None, :] - uj[None, :, :])
            cm = _gelu(_dot(cm1.reshape(N * BJ, D), wc2) + bc2)
            cb = _dot(cm, w1c)
            t3 = cb.reshape(N, BJ, D) + a[:, None, :] + bvj[None, :, :]
            e = _gelu(_dot(_gelu(t3).reshape(N * BJ, D), w2) + b2)
            acc = acc + e.reshape(N, BJ, D).sum(axis=1)

        o1 = _gelu(_dot(hn, wn1a) + _dot(acc, wn1b) + bn1)
        h = _dot(o1, wn2) + bn2
    out_ref[...] = h


def kernel(h, coord, params):
    layers = params["layers"]

    def stack(f):
        return jnp.stack([f(p) for p in layers])

    lng = stack(lambda p: p["ln_g"].reshape(1, D))
    lnb = stack(lambda p: p["ln_b"].reshape(1, D))
    wc1 = stack(lambda p: p["coord1"]["W"])
    bc1 = stack(lambda p: p["coord1"]["b"].reshape(1, D))
    wc2 = stack(lambda p: p["coord2"]["W"])
    bc2 = stack(lambda p: p["coord2"]["b"].reshape(1, D))
    w1a = stack(lambda p: p["edge1"]["W"][0:D])
    w1b = stack(lambda p: p["edge1"]["W"][D:2 * D])
    w1c = stack(lambda p: p["edge1"]["W"][2 * D:3 * D])
    b1 = stack(lambda p: p["edge1"]["b"].reshape(1, D))
    w2 = stack(lambda p: p["edge2"]["W"])
    b2 = stack(lambda p: p["edge2"]["b"].reshape(1, D))
    wn1a = stack(lambda p: p["node1"]["W"][0:D])
    wn1b = stack(lambda p: p["node1"]["W"][D:2 * D])
    bn1 = stack(lambda p: p["node1"]["b"].reshape(1, D))
    wn2 = stack(lambda p: p["node2"]["W"])
    bn2 = stack(lambda p: p["node2"]["b"].reshape(1, D))

    def full(shape):
        return pl.BlockSpec(shape, lambda b: (0,) * len(shape))

    out = pl.pallas_call(
        _gnn_kernel,
        grid=(B,),
        in_specs=[
            pl.BlockSpec((N, D), lambda b: (b, 0)),
            pl.BlockSpec((1, N, 2), lambda b: (b, 0, 0)),
            full((LAYERS, 1, D)), full((LAYERS, 1, D)),
            full((LAYERS, 2, D)), full((LAYERS, 1, D)),
            full((LAYERS, D, D)), full((LAYERS, 1, D)),
            full((LAYERS, D, D)), full((LAYERS, D, D)),
            full((LAYERS, D, D)), full((LAYERS, 1, D)),
            full((LAYERS, D, D)), full((LAYERS, 1, D)),
            full((LAYERS, D, D)), full((LAYERS, D, D)),
            full((LAYERS, 1, D)),
            full((LAYERS, D, D)), full((LAYERS, 1, D)),
        ],
        out_specs=pl.BlockSpec((N, D), lambda b: (b, 0)),
        out_shape=jax.ShapeDtypeStruct((B * N, D), jnp.float32),
    )(h, coord, lng, lnb, wc1, bc1, wc2, bc2, w1a, w1b, w1c, b1, w2, b2,
      wn1a, wn1b, bn1, wn2, bn2)
    return out


# speedup vs baseline: 1.4997x; 1.0088x over previous
import jax
import jax.numpy as jnp
from jax.experimental import pallas as pl

B = 8
N = 256
D = 128
BJ = 256
NJ = N // BJ
LAYERS = 2

_SQRT_HALF = 0.7071067811865476


def _gelu(x):
    return 0.5 * x * (1.0 + jax.lax.erf(x * _SQRT_HALF))


def _dot(x, w):
    return jax.lax.dot_general(x, w, (((1,), (0,)), ((), ())),
                               preferred_element_type=jnp.float32)


def _gnn_kernel(h_ref, coord_ref, lng_ref, lnb_ref, wc1_ref, bc1_ref,
                wc2_ref, bc2_ref, w1a_ref, w1b_ref, w1c_ref, b1_ref,
                w2_ref, b2_ref, wn1a_ref, wn1b_ref, bn1_ref, wn2_ref,
                bn2_ref, out_ref):
    h = h_ref[...]
    c = coord_ref[0]
    cx = c[:, 0:1]
    cy = c[:, 1:2]
    for l in range(LAYERS):
        g = lng_ref[l]
        beta = lnb_ref[l]
        wc1 = wc1_ref[l]
        bc1 = bc1_ref[l]
        wc2 = wc2_ref[l]
        bc2 = bc2_ref[l]
        w1a = w1a_ref[l]
        w1b = w1b_ref[l]
        w1c = w1c_ref[l]
        b1 = b1_ref[l]
        w2 = w2_ref[l]
        b2 = b2_ref[l]
        wn1a = wn1a_ref[l]
        wn1b = wn1b_ref[l]
        bn1 = bn1_ref[l]
        wn2 = wn2_ref[l]
        bn2 = bn2_ref[l]

        mu = jnp.mean(h, axis=1, keepdims=True)
        xc = h - mu
        var = jnp.mean(xc * xc, axis=1, keepdims=True)
        hn = xc * jax.lax.rsqrt(var + 1e-5) * g + beta

        a = _dot(hn, w1a) + b1
        bv = _dot(hn, w1b)
        u_i = cx * wc1[0:1, :] + cy * wc1[1:2, :] + bc1
        u_j = u_i - bc1

        cm0 = _gelu(_dot(_gelu(bc1), wc2) + bc2)
        c0 = _dot(cm0, w1c)
        tii = _gelu(a + bv + c0)
        eii = _gelu(_dot(tii, w2) + b2)

        acc = -eii
        for j in range(NJ):
            uj = u_j[j * BJ:(j + 1) * BJ, :]
            bvj = bv[j * BJ:(j + 1) * BJ, :]
            cm1 = _gelu(u_i[:, None, :] - uj[None, :, :])
            cm = _gelu(_dot(cm1.reshape(N * BJ, D), wc2) + bc2)
            cb = _dot(cm, w1c)
            t3 = cb.reshape(N, BJ, D) + a[:, None, :] + bvj[None, :, :]
            e = _gelu(_dot(_gelu(t3).reshape(N * BJ, D), w2) + b2)
            acc = acc + e.reshape(N, BJ, D).sum(axis=1)

        o1 = _gelu(_dot(hn, wn1a) + _dot(acc, wn1b) + bn1)
        h = _dot(o1, wn2) + bn2
    out_ref[...] = h


def kernel(h, coord, params):
    layers = params["layers"]

    def stack(f):
        return jnp.stack([f(p) for p in layers])

    lng = stack(lambda p: p["ln_g"].reshape(1, D))
    lnb = stack(lambda p: p["ln_b"].reshape(1, D))
    wc1 = stack(lambda p: p["coord1"]["W"])
    bc1 = stack(lambda p: p["coord1"]["b"].reshape(1, D))
    wc2 = stack(lambda p: p["coord2"]["W"])
    bc2 = stack(lambda p: p["coord2"]["b"].reshape(1, D))
    w1a = stack(lambda p: p["edge1"]["W"][0:D])
    w1b = stack(lambda p: p["edge1"]["W"][D:2 * D])
    w1c = stack(lambda p: p["edge1"]["W"][2 * D:3 * D])
    b1 = stack(lambda p: p["edge1"]["b"].reshape(1, D))
    w2 = stack(lambda p: p["edge2"]["W"])
    b2 = stack(lambda p: p["edge2"]["b"].reshape(1, D))
    wn1a = stack(lambda p: p["node1"]["W"][0:D])
    wn1b = stack(lambda p: p["node1"]["W"][D:2 * D])
    bn1 = stack(lambda p: p["node1"]["b"].reshape(1, D))
    wn2 = stack(lambda p: p["node2"]["W"])
    bn2 = stack(lambda p: p["node2"]["b"].reshape(1, D))

    def full(shape):
        return pl.BlockSpec(shape, lambda b: (0,) * len(shape))

    out = pl.pallas_call(
        _gnn_kernel,
        grid=(B,),
        in_specs=[
            pl.BlockSpec((N, D), lambda b: (b, 0)),
            pl.BlockSpec((1, N, 2), lambda b: (b, 0, 0)),
            full((LAYERS, 1, D)), full((LAYERS, 1, D)),
            full((LAYERS, 2, D)), full((LAYERS, 1, D)),
            full((LAYERS, D, D)), full((LAYERS, 1, D)),
            full((LAYERS, D, D)), full((LAYERS, D, D)),
            full((LAYERS, D, D)), full((LAYERS, 1, D)),
            full((LAYERS, D, D)), full((LAYERS, 1, D)),
            full((LAYERS, D, D)), full((LAYERS, D, D)),
            full((LAYERS, 1, D)),
            full((LAYERS, D, D)), full((LAYERS, 1, D)),
        ],
        out_specs=pl.BlockSpec((N, D), lambda b: (b, 0)),
        out_shape=jax.ShapeDtypeStruct((B * N, D), jnp.float32),
    )(h, coord, lng, lnb, wc1, bc1, wc2, bc2, w1a, w1b, w1c, b1, w2, b2,
      wn1a, wn1b, bn1, wn2, bn2)
    return out
